# Initial kernel scaffold; baseline (speedup 1.0000x reference)
import jax
import jax.numpy as jnp
from jax.experimental import pallas as pl


def kernel(inputs, embeddings):
    raise NotImplementedError("write your pallas kernel here")


# speedup vs baseline: 1.6838x; 1.6838x over previous
import functools

import jax
import jax.numpy as jnp
from jax import lax
from jax.experimental import pallas as pl
from jax.experimental.pallas import tpu as pltpu
from jax.experimental.pallas import tpu_sc as plsc

VOCAB = 1000000
EMBED = 64

NC = 2
NS = 16
NW = NC * NS

CHUNK = 128
B_TOTAL = 16384 * 50
N_IDX_ROWS = B_TOTAL // CHUNK
ROWS_PER_W = N_IDX_ROWS // NW


def _sc_gather(idx2d, table):
    mesh = plsc.VectorSubcoreMesh(
        core_axis_name="c", subcore_axis_name="s", num_cores=NC,
        num_subcores=NS)

    @functools.partial(
        pl.kernel,
        out_type=jax.ShapeDtypeStruct((B_TOTAL, EMBED), jnp.float32),
        mesh=mesh,
        compiler_params=pltpu.CompilerParams(use_tc_tiling_on_sc=False),
        scratch_types=[
            pltpu.VMEM((ROWS_PER_W, CHUNK), jnp.int32),
            pltpu.VMEM((CHUNK, EMBED), jnp.float32),
            pltpu.SemaphoreType.DMA,
        ],
    )
    def k(idx_hbm, table_hbm, out_hbm, idx_v, rows_v, sem):
        wid = lax.axis_index("s") * NC + lax.axis_index("c")
        idx_base = wid * ROWS_PER_W
        pltpu.sync_copy(idx_hbm.at[pl.ds(idx_base, ROWS_PER_W)], idx_v)

        def body(j):
            pltpu.async_copy(table_hbm.at[idx_v.at[j]], rows_v, sem).wait()
            out_base = (idx_base + j) * CHUNK
            pltpu.sync_copy(rows_v, out_hbm.at[pl.ds(out_base, CHUNK)])

        pl.loop(0, ROWS_PER_W)(body)

    return k(idx2d, table)


def kernel(inputs, embeddings):
    idx2d = inputs.reshape(N_IDX_ROWS, CHUNK).astype(jnp.int32)
    out = _sc_gather(idx2d, embeddings)
    return out.reshape(inputs.shape[0], inputs.shape[1], EMBED)


# speedup vs baseline: 1.8786x; 1.1157x over previous
import functools

import jax
import jax.numpy as jnp
from jax import lax
from jax.experimental import pallas as pl
from jax.experimental.pallas import tpu as pltpu
from jax.experimental.pallas import tpu_sc as plsc

VOCAB = 1000000
EMBED = 64

NC = 2
NS = 16
NW = NC * NS

CHUNK = 128
B_TOTAL = 16384 * 50
N_IDX_ROWS = B_TOTAL // CHUNK
ROWS_PER_W = N_IDX_ROWS // NW
NBUF = 8


def _sc_gather(idx2d, table):
    mesh = plsc.VectorSubcoreMesh(
        core_axis_name="c", subcore_axis_name="s", num_cores=NC,
        num_subcores=NS)

    @functools.partial(
        pl.kernel,
        out_type=jax.ShapeDtypeStruct((B_TOTAL, EMBED), jnp.float32),
        mesh=mesh,
        compiler_params=pltpu.CompilerParams(use_tc_tiling_on_sc=False),
        scratch_types=[
            pltpu.VMEM((ROWS_PER_W, CHUNK), jnp.int32),
            pltpu.VMEM((NBUF, CHUNK, EMBED), jnp.float32),
            pltpu.SemaphoreType.DMA,
            pltpu.SemaphoreType.DMA,
        ],
    )
    def k(idx_hbm, table_hbm, out_hbm, idx_v, rows_v, gsem, ssem):
        wid = lax.axis_index("s") * NC + lax.axis_index("c")
        idx_base = wid * ROWS_PER_W
        pltpu.sync_copy(idx_hbm.at[pl.ds(idx_base, ROWS_PER_W)], idx_v)

        def gather(j, b):
            pltpu.async_copy(table_hbm.at[idx_v.at[j]], rows_v.at[b], gsem)

        def store(j, b):
            pltpu.async_copy(
                rows_v.at[b], out_hbm.at[pl.ds((idx_base + j) * CHUNK, CHUNK)],
                ssem)

        def wait_gather(b):
            pltpu.make_async_copy(
                table_hbm.at[idx_v.at[0]], rows_v.at[b], gsem).wait()

        def wait_store(b):
            pltpu.make_async_copy(
                rows_v.at[b], out_hbm.at[pl.ds(0, CHUNK)], ssem).wait()

        for b in range(NBUF):
            gather(b, b)
        for b in range(NBUF):
            wait_gather(b)
            store(b, b)

        def body(g):
            for b in range(NBUF):
                wait_store(b)
                gather(g + b, b)
            for b in range(NBUF):
                wait_gather(b)
                store(g + b, b)

        pl.loop(NBUF, ROWS_PER_W, step=NBUF)(body)

        for b in range(NBUF):
            wait_store(b)

    return k(idx2d, table)


def kernel(inputs, embeddings):
    idx2d = inputs.reshape(N_IDX_ROWS, CHUNK).astype(jnp.int32)
    out = _sc_gather(idx2d, embeddings)
    return out.reshape(inputs.shape[0], inputs.shape[1], EMBED)
